# baseline (speedup 1.0000x reference)
import jax
import jax.numpy as jnp
from jax.experimental import pallas as pl
from jax.experimental.pallas import tpu as pltpu


def _chamfer_kernel(x_ref, y_ref, out_ref):
    b = pl.program_id(0)
    x = x_ref[0]
    y = y_ref[0]
    xsq = jnp.sum(x * x, axis=1)
    ysq = jnp.sum(y * y, axis=1)
    dot = jax.lax.dot_general(
        x, y, (((1,), (1,)), ((), ())),
        preferred_element_type=jnp.float32,
        precision=jax.lax.Precision.HIGHEST,
    )
    d2 = xsq[:, None] + ysq[None, :] - 2.0 * dot
    d2 = jnp.maximum(d2, 0.0)
    min_x = jnp.min(d2, axis=1)
    min_y = jnp.min(d2, axis=0)
    partial = jnp.sum(jnp.sqrt(1e-6 + min_x)) + jnp.sum(jnp.sqrt(1e-6 + min_y))

    @pl.when(b == 0)
    def _():
        out_ref[0, 0] = 0.0

    out_ref[0, 0] += partial


def kernel(x, y):
    bs, n, _ = x.shape
    total = pl.pallas_call(
        _chamfer_kernel,
        grid=(bs,),
        in_specs=[
            pl.BlockSpec((1, n, 3), lambda b: (b, 0, 0)),
            pl.BlockSpec((1, n, 3), lambda b: (b, 0, 0)),
        ],
        out_specs=pl.BlockSpec(memory_space=pltpu.SMEM),
        out_shape=jax.ShapeDtypeStruct((1, 1), jnp.float32),
    )(x, y)
    return total[0, 0] / (bs * n)


# speedup vs baseline: 1.0616x; 1.0616x over previous
import jax
import jax.numpy as jnp
from jax.experimental import pallas as pl
from jax.experimental.pallas import tpu as pltpu


def _chamfer_kernel(x_ref, y_ref, out_ref):
    b = pl.program_id(0)
    x = x_ref[0]
    y = y_ref[0]
    n = x.shape[0]
    xsq = jnp.sum(x * x, axis=1, keepdims=True)
    ysq = jnp.sum(y * y, axis=1, keepdims=True)
    ones = jnp.ones((n, 1), dtype=jnp.float32)
    xa = jnp.concatenate([x, ones, xsq], axis=1)
    ya = jnp.concatenate([-2.0 * y, ysq, ones], axis=1)
    d2 = jax.lax.dot_general(
        xa, ya, (((1,), (1,)), ((), ())),
        preferred_element_type=jnp.float32,
        precision=jax.lax.Precision.HIGHEST,
    )
    d2 = jnp.maximum(d2, 0.0)
    min_x = jnp.min(d2, axis=1)
    min_y = jnp.min(d2, axis=0)
    partial = jnp.sum(jnp.sqrt(1e-6 + min_x)) + jnp.sum(jnp.sqrt(1e-6 + min_y))

    @pl.when(b == 0)
    def _():
        out_ref[0, 0] = 0.0

    out_ref[0, 0] += partial


def kernel(x, y):
    bs, n, _ = x.shape
    total = pl.pallas_call(
        _chamfer_kernel,
        grid=(bs,),
        in_specs=[
            pl.BlockSpec((1, n, 3), lambda b: (b, 0, 0)),
            pl.BlockSpec((1, n, 3), lambda b: (b, 0, 0)),
        ],
        out_specs=pl.BlockSpec(memory_space=pltpu.SMEM),
        out_shape=jax.ShapeDtypeStruct((1, 1), jnp.float32),
    )(x, y)
    return total[0, 0] / (bs * n)


# speedup vs baseline: 2.1732x; 2.0471x over previous
import jax
import jax.numpy as jnp
from jax.experimental import pallas as pl
from jax.experimental.pallas import tpu as pltpu


def _split3(a):
    hi = a.astype(jnp.bfloat16).astype(jnp.float32)
    r = a - hi
    mid = r.astype(jnp.bfloat16).astype(jnp.float32)
    lo = r - mid
    return hi, mid, lo


def _chamfer_kernel(x_ref, y_ref, out_ref):
    b = pl.program_id(0)
    x = x_ref[0]
    y = y_ref[0]
    n = x.shape[0]
    xsq = jnp.sum(x * x, axis=1, keepdims=True)
    ysq = jnp.sum(y * y, axis=1, keepdims=True)
    ones = jnp.ones((n, 1), dtype=jnp.float32)
    xa = jnp.concatenate([x, ones, xsq], axis=1)
    ya = jnp.concatenate([-2.0 * y, ysq, ones], axis=1)
    xs = _split3(xa)
    ys3 = _split3(ya)
    acat = jnp.concatenate([xs[i] for i in range(3) for _ in range(3)], axis=1)
    bcat = jnp.concatenate([ys3[j] for _ in range(3) for j in range(3)], axis=1)
    d2 = jax.lax.dot_general(
        acat, bcat, (((1,), (1,)), ((), ())),
        preferred_element_type=jnp.float32,
    )
    d2 = jnp.maximum(d2, 0.0)
    min_x = jnp.min(d2, axis=1)
    min_y = jnp.min(d2, axis=0)
    partial = jnp.sum(jnp.sqrt(1e-6 + min_x)) + jnp.sum(jnp.sqrt(1e-6 + min_y))

    @pl.when(b == 0)
    def _():
        out_ref[0, 0] = 0.0

    out_ref[0, 0] += partial


def kernel(x, y):
    bs, n, _ = x.shape
    total = pl.pallas_call(
        _chamfer_kernel,
        grid=(bs,),
        in_specs=[
            pl.BlockSpec((1, n, 3), lambda b: (b, 0, 0)),
            pl.BlockSpec((1, n, 3), lambda b: (b, 0, 0)),
        ],
        out_specs=pl.BlockSpec(memory_space=pltpu.SMEM),
        out_shape=jax.ShapeDtypeStruct((1, 1), jnp.float32),
    )(x, y)
    return total[0, 0] / (bs * n)
